# Initial kernel scaffold; baseline (speedup 1.0000x reference)
import jax
import jax.numpy as jnp
from jax.experimental import pallas as pl


def kernel(feat0, feat1, feat2, conv_w, conv_b, cls_w, cls_b, reg_w, reg_b):
    raise NotImplementedError("write your pallas kernel here")


# speedup vs baseline: 10.7680x; 10.7680x over previous
import functools

import numpy as np
import jax
import jax.numpy as jnp
from jax.experimental import pallas as pl
from jax.experimental.pallas import tpu as pltpu

_STRIDES = (8, 16, 32)
_SIZES = ((64, 64), (32, 32), (16, 16))
_SCALES = ((1024.0,), (4096.0,), (16384.0,))
_RATIOS = ((0.5, 1.0, 2.0), (0.5, 1.0, 2.0), (0.5, 1.0, 2.0))
_IMG = 512.0
_NMS_THRESH = 0.7
_PER_LEVEL = 333
_BS = 2
_CLIP = float(np.log(1000.0 / 16.0))

_NMAX = 12288
_KW = 336
_MW = 1024
_ROWS = 8


def _anchors_for_level(lvl):
    stride = _STRIDES[lvl]
    H, W = _SIZES[lvl]
    ys = (np.arange(H, dtype=np.float32) + 0.5) * stride
    xs = (np.arange(W, dtype=np.float32) + 0.5) * stride
    cy, cx = np.meshgrid(ys, xs, indexing="ij")
    whs = []
    for s in _SCALES[lvl]:
        for r in _RATIOS[lvl]:
            w = np.sqrt(s / r)
            h = w * r
            whs.append((w, h))
    whs = np.asarray(whs, dtype=np.float32)
    A = whs.shape[0]
    anc = np.zeros((H, W, A, 4), dtype=np.float32)
    for a in range(A):
        w, h = whs[a]
        anc[:, :, a, 0] = cx - w / 2.0
        anc[:, :, a, 1] = cy - h / 2.0
        anc[:, :, a, 2] = cx + w / 2.0
        anc[:, :, a, 3] = cy + h / 2.0
    anc = np.clip(anc, 0.0, _IMG)
    return anc.reshape(-1, 4)


def _packed_anchors():
    out = np.full((4, _ROWS, _NMAX), -1.0, dtype=np.float32)
    for lvl in range(3):
        a = _anchors_for_level(lvl)
        n = a.shape[0]
        for b in range(_BS):
            out[:, 2 * lvl + b, :n] = a.T
    return out


_ANCHORS_NP = _packed_anchors()
_NROW_NP = np.array(
    [[12288], [12288], [3072], [3072], [768], [768], [0], [0]], dtype=np.int32
)


def _conv_heads_kernel(x_ref, w2_ref, wh_ref, cb_ref, hb_ref, out_ref):
    y = jnp.dot(w2_ref[...], x_ref[...], preferred_element_type=jnp.float32)
    y = jnp.maximum(y + cb_ref[...], 0.0)
    p = jnp.dot(wh_ref[...], y, preferred_element_type=jnp.float32)
    out_ref[...] = p + hb_ref[...]


def _nms_kernel(nrow_ref, cls_ref, d0_ref, d1_ref, d2_ref, d3_ref,
                a0_ref, a1_ref, a2_ref, a3_ref,
                out_ref,
                x1_s, y1_s, x2_s, y2_s, s_s, ar_s, act_s,
                mg0, mg1, mg2, mg3, mg4):
    f32 = jnp.float32
    lane = jax.lax.broadcasted_iota(jnp.int32, (_ROWS, _NMAX), 1)

    ax1, ay1, ax2, ay2 = a0_ref[...], a1_ref[...], a2_ref[...], a3_ref[...]
    wa = ax2 - ax1
    ha = ay2 - ay1
    cxa = ax1 + 0.5 * wa
    cya = ay1 + 0.5 * ha
    cx = d0_ref[...] * wa + cxa
    cy = d1_ref[...] * ha + cya
    w = jnp.exp(jnp.minimum(d2_ref[...], _CLIP)) * wa
    h = jnp.exp(jnp.minimum(d3_ref[...], _CLIP)) * ha
    x1 = jnp.clip(cx - 0.5 * w, 0.0, _IMG)
    y1 = jnp.clip(cy - 0.5 * h, 0.0, _IMG)
    x2 = jnp.clip(cx + 0.5 * w, 0.0, _IMG)
    y2 = jnp.clip(cy + 0.5 * h, 0.0, _IMG)
    x1_s[...] = x1
    y1_s[...] = y1
    x2_s[...] = x2
    y2_s[...] = y2
    s_s[...] = jax.nn.sigmoid(cls_ref[...])
    ar_s[...] = jnp.maximum(x2 - x1, 0.0) * jnp.maximum(y2 - y1, 0.0)
    act_s[...] = (lane < nrow_ref[...]).astype(f32)

    k_lane = jax.lax.broadcasted_iota(jnp.int32, (_ROWS, _KW), 1)
    lane_f = lane.astype(f32)

    def body(t, kept):
        ks, kx1, ky1, kx2, ky2 = kept
        act = act_s[...] > 0.5
        s = s_s[...]
        masked = jnp.where(act, s, -1.0)
        maxv = jnp.max(masked, axis=1, keepdims=True)
        valid = maxv > 0.0
        eq = (masked == maxv) & act
        idx = jnp.min(jnp.where(eq, lane_f, f32(1e9)), axis=1, keepdims=True)
        onehot = lane_f == idx
        bx1 = x1_s[...]
        by1 = y1_s[...]
        bx2 = x2_s[...]
        by2 = y2_s[...]
        sx1 = jnp.sum(jnp.where(onehot, bx1, 0.0), axis=1, keepdims=True)
        sy1 = jnp.sum(jnp.where(onehot, by1, 0.0), axis=1, keepdims=True)
        sx2 = jnp.sum(jnp.where(onehot, bx2, 0.0), axis=1, keepdims=True)
        sy2 = jnp.sum(jnp.where(onehot, by2, 0.0), axis=1, keepdims=True)
        area_i = jnp.maximum(sx2 - sx1, 0.0) * jnp.maximum(sy2 - sy1, 0.0)
        xx1 = jnp.maximum(sx1, bx1)
        yy1 = jnp.maximum(sy1, by1)
        xx2 = jnp.minimum(sx2, bx2)
        yy2 = jnp.minimum(sy2, by2)
        inter = jnp.maximum(xx2 - xx1, 0.0) * jnp.maximum(yy2 - yy1, 0.0)
        denom = jnp.maximum(area_i + ar_s[...] - inter, 1e-9)
        suppress = inter > _NMS_THRESH * denom
        kill = (suppress | onehot) & valid
        act_s[...] = jnp.where(kill, 0.0, act_s[...])
        sel = k_lane == t
        ks = jnp.where(sel, jnp.where(valid, maxv, -1.0), ks)
        kx1 = jnp.where(sel, jnp.where(valid, sx1, -1.0), kx1)
        ky1 = jnp.where(sel, jnp.where(valid, sy1, -1.0), ky1)
        kx2 = jnp.where(sel, jnp.where(valid, sx2, -1.0), kx2)
        ky2 = jnp.where(sel, jnp.where(valid, sy2, -1.0), ky2)
        return ks, kx1, ky1, kx2, ky2

    init = tuple(jnp.full((_ROWS, _KW), -1.0, f32) for _ in range(5))
    ks, kx1, ky1, kx2, ky2 = jax.lax.fori_loop(0, _PER_LEVEL, body, init)

    mgs = (mg0, mg1, mg2, mg3, mg4)
    for q, arr in enumerate((ks, kx1, ky1, kx2, ky2)):
        mgs[q][...] = jnp.full((_ROWS, _MW), -1.0, f32)
        for lvl in range(3):
            for b in range(_BS):
                mgs[q][b:b + 1, lvl * _KW:(lvl + 1) * _KW] = (
                    arr[2 * lvl + b:2 * lvl + b + 1, :])

    ts = jnp.transpose(mg0[...])
    tq = [jnp.transpose(mgs[q][...]) for q in (1, 2, 3, 4)]
    iota_r = jax.lax.broadcasted_iota(jnp.int32, (_MW, _MW), 0)
    iota_c = jax.lax.broadcasted_iota(jnp.int32, (_MW, _MW), 1)
    for b in range(_BS):
        s_row = mg0[b:b + 1, :]
        s_i = jnp.broadcast_to(s_row, (_MW, _MW))
        s_j = jnp.broadcast_to(ts[:, b:b + 1], (_MW, _MW))
        gt = (s_j > s_i).astype(f32)
        tie = ((s_j == s_i) & (iota_r < iota_c)).astype(f32)
        rank = jnp.sum(gt + tie, axis=0, keepdims=True)
        onehot = (iota_r.astype(f32) == jnp.broadcast_to(rank, (_MW, _MW)))
        vals = jnp.concatenate(
            [t[:, b:b + 1] for t in tq] + [jnp.zeros((_MW, 4), f32)],
            axis=1)
        out_ref[b] = jnp.dot(onehot.astype(f32), vals,
                             preferred_element_type=jnp.float32)


def kernel(feat0, feat1, feat2, conv_w, conv_b, cls_w, cls_b, reg_w, reg_b):
    feats = (feat0, feat1, feat2)
    cols = []
    for lvl, feat in enumerate(feats):
        H, W = _SIZES[lvl]
        xp = jnp.pad(feat, ((0, 0), (0, 0), (1, 1), (1, 1)))
        p = jnp.stack(
            [xp[:, :, ky:ky + H, kx:kx + W] for ky in range(3) for kx in range(3)],
            axis=1,
        )
        cols.append(p.reshape(_BS, 2304, H * W))
    x9 = jnp.concatenate(cols, axis=2)
    spos = x9.shape[2]
    x9 = jnp.transpose(x9, (1, 0, 2)).reshape(2304, _BS * spos)

    w2 = jnp.transpose(conv_w, (0, 2, 3, 1)).reshape(256, 2304)
    wh = jnp.concatenate([cls_w.reshape(3, 256), reg_w.reshape(12, 256)], axis=0)
    wh = jnp.pad(wh, ((0, 1), (0, 0)))
    hb = jnp.pad(jnp.concatenate([cls_b, reg_b]), (0, 1)).reshape(16, 1)
    cb = conv_b.reshape(256, 1)

    ncols = x9.shape[1]
    tile = 512
    grid = ncols // tile
    heads = pl.pallas_call(
        _conv_heads_kernel,
        grid=(grid,),
        in_specs=[
            pl.BlockSpec((2304, tile), lambda j: (0, j)),
            pl.BlockSpec((256, 2304), lambda j: (0, 0)),
            pl.BlockSpec((16, 256), lambda j: (0, 0)),
            pl.BlockSpec((256, 1), lambda j: (0, 0)),
            pl.BlockSpec((16, 1), lambda j: (0, 0)),
        ],
        out_specs=pl.BlockSpec((16, tile), lambda j: (0, j)),
        out_shape=jax.ShapeDtypeStruct((16, ncols), jnp.float32),
    )(x9, w2, wh, cb, hb)

    pr = heads.reshape(16, _BS, spos)
    offs = (0, 4096, 5120)
    cls8 = jnp.full((_ROWS, _NMAX), 0.0, jnp.float32)
    d8 = [jnp.zeros((_ROWS, _NMAX), jnp.float32) for _ in range(4)]
    for lvl in range(3):
        H, W = _SIZES[lvl]
        S = H * W
        pl_blk = pr[:, :, offs[lvl]:offs[lvl] + S]
        cls_l = jnp.transpose(pl_blk[0:3], (1, 2, 0)).reshape(_BS, S * 3)
        reg_l = jnp.transpose(
            pl_blk[3:15].reshape(3, 4, _BS, S), (1, 2, 3, 0)
        ).reshape(4, _BS, S * 3)
        r = 2 * lvl
        cls8 = cls8.at[r:r + 2, : S * 3].set(cls_l)
        for k in range(4):
            d8[k] = d8[k].at[r:r + 2, : S * 3].set(reg_l[k])

    anc = jnp.asarray(_ANCHORS_NP)
    nrow = jnp.asarray(_NROW_NP)

    out = pl.pallas_call(
        _nms_kernel,
        out_shape=jax.ShapeDtypeStruct((_BS, _MW, 8), jnp.float32),
        scratch_shapes=[pltpu.VMEM((_ROWS, _NMAX), jnp.float32)] * 7
        + [pltpu.VMEM((_ROWS, _MW), jnp.float32)] * 5,
    )(nrow, cls8, d8[0], d8[1], d8[2], d8[3], anc[0], anc[1], anc[2], anc[3])

    return out[:, :999, :4]
